# baseline (speedup 1.0000x reference)
import functools

import jax
import jax.numpy as jnp
from jax import lax
from jax.experimental import pallas as pl
from jax.experimental.pallas import tpu as pltpu
from jax.experimental.pallas import tpu_sc as plsc

B, L, D = 4096, 50, 32
V = 1000000
NC, NS = 2, 16
NW = NC * NS
CH = 512
NCH = V // CH
TAIL_OFF = NCH * CH
SR = NCH * 128
SRF = SR + 64
CPG = 63
NG = NCH // CPG
RPW = B * L // NW

_mesh = plsc.VectorSubcoreMesh(
    core_axis_name="c", subcore_axis_name="s", num_cores=NC, num_subcores=NS
)


def _tc_repack_body(in_ref, out_ref):
    for g in range(CPG):
        x = jnp.concatenate(
            [in_ref[:, g * CH + 128 * a:g * CH + 128 * (a + 1)]
             for a in range(4)],
            axis=0,
        )
        out_ref[128 * g:128 * (g + 1), :] = jnp.transpose(x, (1, 0))


_tc_repack = pl.pallas_call(
    _tc_repack_body,
    grid=(NG,),
    in_specs=[pl.BlockSpec((D, CPG * CH), lambda c: (0, c))],
    out_specs=pl.BlockSpec((CPG * 128, 128), lambda c: (c, 0)),
    out_shape=jax.ShapeDtypeStruct((SRF, 128), jnp.float32),
)


@functools.partial(
    pl.kernel,
    out_type=jax.ShapeDtypeStruct((L, D // 8, B // 128, 8, 128), jnp.float32),
    mesh=_mesh,
    scratch_types=(
        [pltpu.VMEM((RPW,), jnp.int32)]
        + [pltpu.VMEM((128, D), jnp.float32) for _ in range(5)]
        + [pltpu.VMEM((4, 1, 8, 128), jnp.float32) for _ in range(2)]
        + [pltpu.SemaphoreType.DMA for _ in range(7)]
    ),
    compiler_params=pltpu.CompilerParams(
        use_tc_tiling_on_sc=False, needs_layout_passes=False
    ),
)
def _sc_gather(gidx_hbm, packed_hbm, out_hbm, idx_v, *rest):
    gbuf = rest[0:5]
    obuf = rest[5:7]
    gsem = rest[7:12]
    osem = rest[12:14]

    cid = lax.axis_index("c")
    sid = lax.axis_index("s")
    wid = sid * NC + cid
    iota = lax.iota(jnp.int32, 16)

    pltpu.sync_copy(gidx_hbm.at[pl.ds(wid * RPW, RPW)], idx_v)

    rows_e = [iota + 16 * e for e in range(8)]
    cols_d = [jnp.full((16,), d, jnp.int32) for d in range(D)]

    def start_gather(l, b):
        idx_slice = idx_v.at[pl.ds(l * 128, 128)]
        pltpu.async_copy(packed_hbm.at[idx_slice], gbuf[b], gsem[b])

    for b in range(5):
        start_gather(b, b)

    @pl.loop(0, L, step=5)
    def _(l0):
        for b in range(5):
            l = l0 + b
            ob = b % 2
            idx_slice = idx_v.at[pl.ds(l * 128, 128)]
            pltpu.make_async_copy(
                packed_hbm.at[idx_slice], gbuf[b], gsem[b]
            ).wait()
            dst = out_hbm.at[l, :, pl.ds(wid, 1), :, :]

            @pl.when(l >= 2)
            def _():
                pltpu.make_async_copy(obuf[ob], dst, osem[ob]).wait()

            for e in range(8):
                for d in range(D):
                    vals = plsc.load_gather(gbuf[b], [rows_e[e], cols_d[d]])
                    obuf[ob][d // 8, 0, d % 8, 16 * e:16 * (e + 1)] = vals

            pltpu.async_copy(obuf[ob], dst, osem[ob])

            @pl.when(l + 5 < L)
            def _():
                start_gather(l + 5, b)

    for lf, ob in ((L - 2, 1), (L - 1, 0)):
        dst = out_hbm.at[lf, :, pl.ds(wid, 1), :, :]
        pltpu.make_async_copy(obuf[ob], dst, osem[ob]).wait()


def kernel(word_inputs, feature_inputs, word_seq_lengths, char_inputs,
           char_seq_lengths, char_seq_recover, word_table,
           feat_table_0, feat_table_1):
    table_t = word_table.T
    packed = _tc_repack(table_t)
    tail = jnp.concatenate(
        [word_table[TAIL_OFF:V, :],
         jnp.zeros((V - TAIL_OFF, 128 - D), jnp.float32)],
        axis=1,
    )
    packed = lax.dynamic_update_slice(packed, tail, (SR, 0))
    packed4 = packed.reshape(SRF * 4, D)

    w = word_inputs.astype(jnp.int32).T
    r32 = (
        lax.shift_left(lax.shift_right_logical(w, 9), 9)
        + lax.shift_left(w & 127, 2)
        + (lax.shift_right_logical(w, 7) & 3)
    )
    gidx = r32.reshape(L, NW, 128).transpose(1, 0, 2).reshape(-1)

    out5 = _sc_gather(gidx, packed4)
    return out5.transpose(2, 4, 0, 1, 3).reshape(B, L, D)


# speedup vs baseline: 1.0219x; 1.0219x over previous
import functools

import jax
import jax.numpy as jnp
from jax import lax
from jax.experimental import pallas as pl
from jax.experimental.pallas import tpu as pltpu
from jax.experimental.pallas import tpu_sc as plsc

B, L, D = 4096, 50, 32
V = 1000000
NC, NS = 2, 16
NW = NC * NS
CH = 512
NCH = V // CH
TAIL_OFF = NCH * CH
SR = NCH * 128
SRF = SR + 64
CPG = 93
NG = NCH // CPG

_mesh = plsc.VectorSubcoreMesh(
    core_axis_name="c", subcore_axis_name="s", num_cores=NC, num_subcores=NS
)


def _tc_repack_body(in_ref, out_ref):
    for g in range(CPG):
        x = jnp.concatenate(
            [in_ref[:, g * CH + 128 * a:g * CH + 128 * (a + 1)]
             for a in range(4)],
            axis=0,
        )
        out_ref[128 * g:128 * (g + 1), :] = jnp.transpose(x, (1, 0))


_tc_repack = pl.pallas_call(
    _tc_repack_body,
    grid=(NG,),
    in_specs=[pl.BlockSpec((D, CPG * CH), lambda c: (0, c))],
    out_specs=pl.BlockSpec((CPG * 128, 128), lambda c: (c, 0)),
    out_shape=jax.ShapeDtypeStruct((SRF, 128), jnp.float32),
)


@functools.partial(
    pl.kernel,
    out_type=jax.ShapeDtypeStruct((L, D, B), jnp.float32),
    mesh=_mesh,
    scratch_types=(
        [pltpu.VMEM((D, 64), jnp.float32), pltpu.VMEM((64, 128), jnp.float32)]
        + [pltpu.VMEM((L, 128), jnp.int32) for _ in range(3)]
        + [pltpu.VMEM((128, 128), jnp.float32) for _ in range(5)]
        + [pltpu.VMEM((D, 128), jnp.float32) for _ in range(2)]
        + [pltpu.SemaphoreType.DMA for _ in range(7)]
        + [pltpu.SemaphoreType.REGULAR]
    ),
    compiler_params=pltpu.CompilerParams(
        use_tc_tiling_on_sc=True, needs_layout_passes=False
    ),
)
def _sc_gather(idx_hbm, table_hbm, packed_hbm, out_hbm, tin, tout,
               idx_v, gidx, wm32, *rest):
    gbuf = rest[0:5]
    obuf = rest[5:7]
    gsem = rest[7:12]
    osem = rest[12:14]
    xsem = rest[14]

    cid = lax.axis_index("c")
    sid = lax.axis_index("s")
    wid = sid * NC + cid
    iota = lax.iota(jnp.int32, 16)

    @pl.when(sid == 0)
    def _():
        pltpu.sync_copy(table_hbm.at[:, pl.ds(TAIL_OFF, 64)], tin)
        for e in range(2):
            rows = iota + 16 * e
            for t in range(64):
                cols = jnp.full((16,), t, jnp.int32)
                vals = plsc.load_gather(tin, [rows, cols])
                tout[t, 16 * e:16 * (e + 1)] = vals
        pltpu.sync_copy(tout, packed_hbm.at[pl.ds(SR, 64)])

    plsc.subcore_barrier()

    @pl.when(sid == 0)
    def _():
        pl.semaphore_signal(xsem, 1, core_index=1 - cid)
        pl.semaphore_wait(xsem, 1)

    plsc.subcore_barrier()

    b0 = pl.multiple_of(wid * 128, 128)

    pltpu.sync_copy(idx_hbm.at[:, pl.ds(b0, 128)], idx_v)

    @pl.loop(0, L)
    def _(l):
        for e in range(8):
            w = idx_v[l, 16 * e:16 * (e + 1)]
            r = lax.shift_left(lax.shift_right_logical(w, 9), 7) + (w & 127)
            gidx[l, 16 * e:16 * (e + 1)] = r
            wm32[l, 16 * e:16 * (e + 1)] = lax.shift_left(
                lax.shift_right_logical(w, 7) & 3, 5
            )

    def start_gather(l, b):
        pltpu.async_copy(packed_hbm.at[gidx.at[l]], gbuf[b], gsem[b])

    for b in range(5):
        start_gather(b, b)

    @pl.loop(0, L, step=5)
    def _(l0):
        for b in range(5):
            l = l0 + b
            ob = b % 2
            pltpu.make_async_copy(
                packed_hbm.at[gidx.at[l]], gbuf[b], gsem[b]
            ).wait()
            dst = out_hbm.at[l, :, pl.ds(b0, 128)]

            @pl.when(l >= 2)
            def _():
                pltpu.make_async_copy(obuf[ob], dst, osem[ob]).wait()

            for e in range(8):
                rows = iota + 16 * e
                wrow = wm32[l, 16 * e:16 * (e + 1)]
                for d in range(32):
                    vals = plsc.load_gather(gbuf[b], [rows, wrow + d])
                    obuf[ob][d, 16 * e:16 * (e + 1)] = vals

            pltpu.async_copy(obuf[ob], dst, osem[ob])

            @pl.when(l + 5 < L)
            def _():
                start_gather(l + 5, b)

    for lf, ob in ((L - 2, 1), (L - 1, 0)):
        dst = out_hbm.at[lf, :, pl.ds(b0, 128)]
        pltpu.make_async_copy(obuf[ob], dst, osem[ob]).wait()


def kernel(word_inputs, feature_inputs, word_seq_lengths, char_inputs,
           char_seq_lengths, char_seq_recover, word_table,
           feat_table_0, feat_table_1):
    table_t = word_table.T
    packed_ref = jax.new_ref(_tc_repack(table_t))
    out = _sc_gather(word_inputs.astype(jnp.int32).T, table_t, packed_ref)
    return out.transpose(2, 0, 1)
